# baseline (speedup 1.0000x reference)
import functools

import jax
import jax.numpy as jnp
from jax import lax
from jax.experimental import pallas as pl
from jax.experimental.pallas import tpu as pltpu
from jax.experimental.pallas import tpu_sc as plsc

_DIM = 64
_B = 4096
_H = 50
_TOTAL = _B * _H
_NW = 32
_PER_W = _TOTAL // _NW
_C = 160
_NC = _PER_W // _C

_mesh = plsc.VectorSubcoreMesh(core_axis_name="c", subcore_axis_name="s")


@functools.partial(
    pl.kernel,
    mesh=_mesh,
    out_type=jax.ShapeDtypeStruct((_TOTAL, _DIM), jnp.float32),
    compiler_params=pltpu.CompilerParams(
        needs_layout_passes=False, skip_device_barrier=True
    ),
    scratch_types=[
        pltpu.VMEM((_PER_W,), jnp.int32),
        pltpu.VMEM((_PER_W,), jnp.int32),
        pltpu.VMEM((2, _C, 128), jnp.float32),
        pltpu.VMEM((2, _C, _DIM), jnp.float32),
        pltpu.SemaphoreType.DMA,
        pltpu.SemaphoreType.DMA,
        pltpu.SemaphoreType.DMA,
        pltpu.SemaphoreType.DMA,
    ],
)
def _gather_w(
    idx_hbm, table_w, out_hbm, idx_v, widx_v, wide_v, outb_v,
    g0, g1, s0, s1,
):
    gsem = (g0, g1)
    ssem = (s0, s1)
    wid = lax.axis_index("s") * 2 + lax.axis_index("c")
    base = wid * _PER_W
    pltpu.sync_copy(idx_hbm.at[pl.ds(base, _PER_W)], idx_v)

    @plsc.parallel_loop(0, _PER_W // 16, step=1, unroll=8)
    def _(i):
        widx_v[pl.ds(i * 16, 16)] = idx_v[pl.ds(i * 16, 16)] >> 1

    def start_gather(p, c):
        pltpu.async_copy(
            table_w.at[widx_v.at[pl.ds(c * _C, _C)]], wide_v.at[p], gsem[p]
        )

    def wait_gather(p, c):
        pltpu.make_async_copy(
            table_w.at[widx_v.at[pl.ds(c * _C, _C)]], wide_v.at[p], gsem[p]
        ).wait()

    def start_store(p, c):
        pltpu.async_copy(
            outb_v.at[p], out_hbm.at[pl.ds(base + c * _C, _C), :], ssem[p]
        )

    def wait_store(p, c):
        pltpu.make_async_copy(
            outb_v.at[p], out_hbm.at[pl.ds(base + c * _C, _C), :], ssem[p]
        ).wait()

    iot = lax.iota(jnp.int32, 16)

    def stitch(p, c):
        wv = wide_v.at[p]
        ob = outb_v.at[p]

        @plsc.parallel_loop(0, _DIM, step=1, unroll=2)
        def _(cc):
            ccv = jnp.full((16,), cc, jnp.int32)
            for g in range(_C // 16):
                r16 = iot + g * 16
                pv = (idx_v[pl.ds(c * _C + g * 16, 16)] & 1) << 6
                v = plsc.load_gather(wv, [r16, pv + ccv])
                plsc.store_scatter(ob, [r16, ccv], v)

    start_gather(0, 0)
    start_gather(1, 1)

    def body(G, carry):
        for p in range(2):
            c = 2 * G + p
            wait_gather(p, c)

            @pl.when(c >= 2)
            def _():
                wait_store(p, c - 2)

            stitch(p, c)
            start_store(p, c)

            @pl.when(c < _NC - 2)
            def _():
                start_gather(p, c + 2)

        return carry

    lax.fori_loop(0, _NC // 2, body, 0)
    wait_store(0, _NC - 2)
    wait_store(1, _NC - 1)


def kernel(model_input, table):
    idx = model_input.T.reshape(-1).astype(jnp.int32)
    table_w = table.reshape(500000, 128)
    out = _gather_w(idx, table_w)
    return out.reshape(_H, _B, _DIM).transpose(1, 0, 2)


# speedup vs baseline: 1.2149x; 1.2149x over previous
import functools

import jax
import jax.numpy as jnp
from jax import lax
from jax.experimental import pallas as pl
from jax.experimental.pallas import tpu as pltpu
from jax.experimental.pallas import tpu_sc as plsc

_DIM = 64
_B = 4096
_H = 50
_TOTAL = _B * _H
_NW = 32
_PER_W = _TOTAL // _NW
_CHUNK = 800
_NCHUNK = _PER_W // _CHUNK

_mesh = plsc.VectorSubcoreMesh(core_axis_name="c", subcore_axis_name="s")


@functools.partial(
    pl.kernel,
    mesh=_mesh,
    out_type=jax.ShapeDtypeStruct((_TOTAL, _DIM), jnp.float32),
    compiler_params=pltpu.CompilerParams(
        use_tc_tiling_on_sc=False, skip_device_barrier=True
    ),
    scratch_types=[
        pltpu.VMEM((_PER_W,), jnp.int32),
        pltpu.VMEM((2, _CHUNK, _DIM), jnp.float32),
        pltpu.SemaphoreType.DMA,
        pltpu.SemaphoreType.DMA,
        pltpu.SemaphoreType.DMA,
        pltpu.SemaphoreType.DMA,
    ],
)
def _gather(idx_hbm, table_hbm, out_hbm, idx_v, rows_v, g0, g1, s0, s1):
    gsem = (g0, g1)
    ssem = (s0, s1)
    wid = lax.axis_index("s") * 2 + lax.axis_index("c")
    base = wid * _PER_W
    pltpu.sync_copy(idx_hbm.at[pl.ds(base, _PER_W)], idx_v)

    def start_gather(c):
        b = c % 2
        pltpu.async_copy(
            table_hbm.at[idx_v.at[pl.ds(c * _CHUNK, _CHUNK)]],
            rows_v.at[b],
            gsem[b],
        )

    def wait_gather(c):
        b = c % 2
        pltpu.make_async_copy(
            table_hbm.at[idx_v.at[pl.ds(c * _CHUNK, _CHUNK)]],
            rows_v.at[b],
            gsem[b],
        ).wait()

    def start_store(c):
        b = c % 2
        pltpu.async_copy(
            rows_v.at[b], out_hbm.at[pl.ds(base + c * _CHUNK, _CHUNK)], ssem[b]
        )

    def wait_store(c):
        b = c % 2
        pltpu.make_async_copy(
            rows_v.at[b], out_hbm.at[pl.ds(base + c * _CHUNK, _CHUNK)], ssem[b]
        ).wait()

    start_gather(0)
    for c in range(1, _NCHUNK):
        if c >= 2:
            wait_store(c - 2)
        start_gather(c)
        wait_gather(c - 1)
        start_store(c - 1)
    wait_gather(_NCHUNK - 1)
    start_store(_NCHUNK - 1)
    wait_store(_NCHUNK - 2)
    wait_store(_NCHUNK - 1)


def kernel(model_input, table):
    idx = model_input.T.reshape(-1).astype(jnp.int32)
    out = _gather(idx, table)
    return out.reshape(_H, _B, _DIM).transpose(1, 0, 2)
